# baseline (speedup 1.0000x reference)
import jax
import jax.numpy as jnp
from jax.experimental import pallas as pl
from jax.experimental.pallas import tpu as pltpu

B = 16
TT = 2048
TH = TT // 2


def _mlp_seg_kernel(cu_ref, x_ref, W1_ref, b1_ref, W2_ref, b2_ref,
                    W3_ref, b3_ref, out_ref, acc_ref, h2a_ref, h2b_ref):
    t = pl.program_id(0)
    nt = pl.num_programs(0)
    lo = jnp.stack([cu_ref[s] for s in range(B)]).reshape(B, 1)
    hi = jnp.stack([cu_ref[s + 1] for s in range(B)]).reshape(B, 1)

    @pl.when(t > 0)
    def _():
        gi = (t - 1) * TT + jax.lax.broadcasted_iota(jnp.int32, (1, TT), 1)
        S = ((gi >= lo) & (gi < hi)).astype(jnp.bfloat16)
        contrib = (
            jnp.dot(S[:, 0:TH], h2a_ref[...], preferred_element_type=jnp.float32)
            + jnp.dot(S[:, TH:TT], h2b_ref[...], preferred_element_type=jnp.float32))

        @pl.when(t == 1)
        def _():
            acc_ref[...] = contrib

        @pl.when(t > 1)
        def _():
            acc_ref[...] += contrib

    @pl.when(t < nt - 1)
    def _():
        xa = x_ref[0:TH, :]
        xb = x_ref[TH:TT, :]
        h1a = jnp.dot(xa, W1_ref[...], preferred_element_type=jnp.float32)
        h1a = jnp.maximum(h1a + b1_ref[...], 0.0)
        h1b = jnp.dot(xb, W1_ref[...], preferred_element_type=jnp.float32)
        h1b = jnp.maximum(h1b + b1_ref[...], 0.0)
        h2a = jnp.dot(h1a, W2_ref[...], preferred_element_type=jnp.float32)
        h2a_ref[...] = jnp.maximum(h2a + b2_ref[...], 0.0).astype(jnp.bfloat16)
        h2b = jnp.dot(h1b, W2_ref[...], preferred_element_type=jnp.float32)
        h2b_ref[...] = jnp.maximum(h2b + b2_ref[...], 0.0).astype(jnp.bfloat16)

    @pl.when(t == nt - 1)
    def _():
        lens = (hi - lo).astype(jnp.float32)
        out_ref[...] = (
            jnp.dot(acc_ref[...], W3_ref[...],
                    preferred_element_type=jnp.float32)
            + lens * b3_ref[...])


@jax.jit
def kernel(flat, cu_seqlens, W1, b1, W2, b2, W3, b3):
    T, D = flat.shape
    H = W1.shape[1]
    O = W3.shape[1]
    nT = T // TT

    grid_spec = pltpu.PrefetchScalarGridSpec(
        num_scalar_prefetch=1,
        grid=(nT + 1,),
        in_specs=[
            pl.BlockSpec((TT, D), lambda t, cu: (jnp.minimum(t, nT - 1), 0)),
            pl.BlockSpec((D, H), lambda t, cu: (0, 0)),
            pl.BlockSpec((1, H), lambda t, cu: (0, 0)),
            pl.BlockSpec((H, H), lambda t, cu: (0, 0)),
            pl.BlockSpec((1, H), lambda t, cu: (0, 0)),
            pl.BlockSpec((H, O), lambda t, cu: (0, 0)),
            pl.BlockSpec((1, O), lambda t, cu: (0, 0)),
        ],
        out_specs=pl.BlockSpec((B, O), lambda t, cu: (0, 0)),
        scratch_shapes=[
            pltpu.VMEM((B, H), jnp.float32),
            pltpu.VMEM((TH, H), jnp.bfloat16),
            pltpu.VMEM((TH, H), jnp.bfloat16),
        ],
    )

    out = pl.pallas_call(
        _mlp_seg_kernel,
        grid_spec=grid_spec,
        out_shape=jax.ShapeDtypeStruct((B, O), jnp.float32),
    )(cu_seqlens, flat, W1, b1.reshape(1, H),
      W2, b2.reshape(1, H), W3, b3.reshape(1, O))
    return out


# speedup vs baseline: 1.0140x; 1.0140x over previous
import jax
import jax.numpy as jnp
from jax.experimental import pallas as pl
from jax.experimental.pallas import tpu as pltpu

B = 16
TT = 2048


def _mlp_seg_kernel(cu_ref, x_ref, W1_ref, b1_ref, W2_ref, b2_ref,
                    W3_ref, b3_ref, out_ref, acc_ref):
    t = pl.program_id(0)
    nt = pl.num_programs(0)

    TH = TT // 2
    xa = x_ref[0:TH, :]
    xb = x_ref[TH:TT, :]
    h1a = jnp.dot(xa, W1_ref[...], preferred_element_type=jnp.float32)
    h1a = jnp.maximum(h1a + b1_ref[...], 0.0)
    h1b = jnp.dot(xb, W1_ref[...], preferred_element_type=jnp.float32)
    h1b = jnp.maximum(h1b + b1_ref[...], 0.0)
    h2a = jnp.dot(h1a, W2_ref[...], preferred_element_type=jnp.float32)
    h2a = jnp.maximum(h2a + b2_ref[...], 0.0)
    h2b = jnp.dot(h1b, W2_ref[...], preferred_element_type=jnp.float32)
    h2b = jnp.maximum(h2b + b2_ref[...], 0.0)

    gi = t * TT + jax.lax.broadcasted_iota(jnp.int32, (1, TT), 1)
    lo = jnp.stack([cu_ref[s] for s in range(B)]).reshape(B, 1)
    hi = jnp.stack([cu_ref[s + 1] for s in range(B)]).reshape(B, 1)
    S = ((gi >= lo) & (gi < hi)).astype(jnp.float32)
    contrib = (jnp.dot(S[:, 0:TH], h2a, preferred_element_type=jnp.float32)
               + jnp.dot(S[:, TH:TT], h2b, preferred_element_type=jnp.float32))

    @pl.when(t == 0)
    def _():
        acc_ref[...] = contrib

    @pl.when(t > 0)
    def _():
        acc_ref[...] += contrib

    @pl.when(t == nt - 1)
    def _():
        lens = (hi - lo).astype(jnp.float32)
        out_ref[...] = (
            jnp.dot(acc_ref[...], W3_ref[...],
                    preferred_element_type=jnp.float32)
            + lens * b3_ref[...])


@jax.jit
def kernel(flat, cu_seqlens, W1, b1, W2, b2, W3, b3):
    T, D = flat.shape
    H = W1.shape[1]
    O = W3.shape[1]
    nT = T // TT

    grid_spec = pltpu.PrefetchScalarGridSpec(
        num_scalar_prefetch=1,
        grid=(nT,),
        in_specs=[
            pl.BlockSpec((TT, D), lambda t, cu: (t, 0)),
            pl.BlockSpec((D, H), lambda t, cu: (0, 0)),
            pl.BlockSpec((1, H), lambda t, cu: (0, 0)),
            pl.BlockSpec((H, H), lambda t, cu: (0, 0)),
            pl.BlockSpec((1, H), lambda t, cu: (0, 0)),
            pl.BlockSpec((H, O), lambda t, cu: (0, 0)),
            pl.BlockSpec((1, O), lambda t, cu: (0, 0)),
        ],
        out_specs=pl.BlockSpec((B, O), lambda t, cu: (0, 0)),
        scratch_shapes=[pltpu.VMEM((B, H), jnp.float32)],
    )

    out = pl.pallas_call(
        _mlp_seg_kernel,
        grid_spec=grid_spec,
        out_shape=jax.ShapeDtypeStruct((B, O), jnp.float32),
    )(cu_seqlens, flat, W1, b1.reshape(1, H),
      W2, b2.reshape(1, H), W3, b3.reshape(1, O))
    return out
